# baseline (speedup 1.0000x reference)
import functools

import jax
import jax.numpy as jnp
from jax import lax
from jax.experimental import pallas as pl
from jax.experimental.pallas import tpu as pltpu
from jax.experimental.pallas import tpu_sc as plsc

BATCH = 128
FEATS = 32768
NC = 2
NS = 16
LANES = 16
NW = NC * NS
ROWS_PER_W = BATCH // NW
UNROLL = 8
VECS = FEATS // LANES
STEPS = VECS // UNROLL

_mesh = plsc.VectorSubcoreMesh(core_axis_name="c", subcore_axis_name="s")


@functools.partial(
    pl.kernel,
    mesh=_mesh,
    compiler_params=pltpu.CompilerParams(needs_layout_passes=False),
    out_type=jax.ShapeDtypeStruct((NW, LANES), jnp.float32),
    scratch_types=[
        pltpu.VMEM((FEATS,), jnp.float32),
        pltpu.VMEM((FEATS,), jnp.float32),
        pltpu.VMEM((LANES,), jnp.float32),
        pltpu.VMEM((LANES,), jnp.float32),
        pltpu.SemaphoreType.DMA,
        pltpu.SemaphoreType.DMA,
    ],
)
def _closest_sc(inp_hbm, prev_hbm, out_hbm, buf_a, buf_b, prev_v, out_v,
                sem_a, sem_b):
    c = lax.axis_index("c")
    s = lax.axis_index("s")
    wid = s * NC + c
    base_row = wid * ROWS_PER_W

    pltpu.sync_copy(prev_hbm.at[wid], prev_v)
    pvec = prev_v[...]

    bufs = (buf_a, buf_b)
    sems = (sem_a, sem_b)
    cur = pltpu.async_copy(inp_hbm.at[base_row], bufs[0], sems[0])

    lane_iota = lax.iota(jnp.int32, LANES)
    res = jnp.zeros((LANES,), jnp.float32)

    for r in range(ROWS_PER_W):
        nxt = None
        if r + 1 < ROWS_PER_W:
            nxt = pltpu.async_copy(
                inp_hbm.at[base_row + r + 1], bufs[(r + 1) % 2],
                sems[(r + 1) % 2])
        cur.wait()
        buf = bufs[r % 2]
        p = pvec[r]

        init = tuple(jnp.full((LANES,), 0xFFFFFFFF, jnp.uint32)
                     for _ in range(UNROLL))

        @plsc.parallel_loop(0, STEPS, carry=init)
        def step(i, carry, buf=buf, p=p):
            new = []
            start = i * (UNROLL * LANES)
            for k in range(UNROLL):
                x = buf[pl.ds(start + k * LANES, LANES)]
                d = x - p
                bits = lax.bitcast_convert_type(d, jnp.uint32)
                key = (bits << 1) | (bits >> 31)
                new.append(jnp.minimum(carry[k], key))
            return tuple(new)

        carry = step
        kmin = carry[0]
        for k in range(1, UNROLL):
            kmin = jnp.minimum(kmin, carry[k])
        ks, _ = plsc.sort_key_val(kmin, kmin)
        dbits = (ks >> 1) | (ks << 31)
        dvec = lax.bitcast_convert_type(dbits, jnp.float32) + p
        val = dvec[0]
        res = jnp.where(lane_iota == r, val, res)
        cur = nxt

    out_v[...] = res
    pltpu.sync_copy(out_v, out_hbm.at[wid])


def kernel(input, prev_output):
    prev_staged = jnp.zeros((NW, LANES), jnp.float32)
    prev_staged = prev_staged.at[:, :ROWS_PER_W].set(
        prev_output.reshape(NW, ROWS_PER_W))
    out = _closest_sc(input, prev_staged)
    return out[:, :ROWS_PER_W].reshape(BATCH, 1)
